# baseline (speedup 1.0000x reference)
import jax
import jax.numpy as jnp
from jax import lax
from jax.experimental import pallas as pl
from jax.experimental.pallas import tpu as pltpu

_NCLS = 3
_EPS = 1e-6
_INF_BITS = 0x7F800000
_CH = 16


def _loss_kernel(gt_ref, ax_ref, ay_ref, by_ref, pred_ref, out_ref,
                 nv_ref, acc_ref):
    b = pl.program_id(0)
    A = nv_ref.shape[0]
    H, W = pred_ref.shape[2], pred_ref.shape[3]
    N = gt_ref.shape[1]

    @pl.when(b == 0)
    def _init():
        for i in range(5):
            acc_ref[i] = 0.0

    v_obj = jnp.zeros((_CH, W), jnp.float32)
    v_np = jnp.zeros((_CH, W), jnp.float32)
    v_neg = jnp.zeros((_CH, W), jnp.float32)
    v_ce = jnp.zeros((_CH, W), jnp.float32)
    v_ll = jnp.zeros((_CH, W), jnp.float32)

    for a in range(A):
        ax1 = ax_ref[0, a:a + 1, :]
        ax2 = ax_ref[1, a:a + 1, :]
        awx = jnp.maximum(ax2 - ax1, 0.0)
        aw = jnp.maximum(ax2 - ax1, _EPS)
        acx = ax1 + 0.5 * aw
        ivaw = 1.0 / (aw + _EPS)
        law = jnp.log(aw + _EPS)

        ay1 = ay_ref[0, :, a:a + 1]
        ay2 = ay_ref[1, :, a:a + 1]
        awy = jnp.maximum(ay2 - ay1, 0.0)
        ah = jnp.maximum(ay2 - ay1, _EPS)
        acy = ay1 + 0.5 * ah
        ivah = 1.0 / (ah + _EPS)
        lah = jnp.log(ah + _EPS)

        by1 = by_ref[0, 0:1, :]
        by2 = by_ref[0, 1:2, :]
        wy_all = jnp.maximum(jnp.minimum(ay2, by2) - jnp.maximum(ay1, by1),
                             0.0)

        wxs = []
        for j in range(N):
            bx1 = gt_ref[b, j, 0]
            bx2 = gt_ref[b, j, 2]
            wxs.append(jnp.maximum(
                jnp.minimum(ax2, bx2) - jnp.maximum(ax1, bx1), 0.0))

        for hc in range(H // _CH):
            r0 = hc * _CH
            rows = pl.ds(r0, _CH)
            area = awy[r0:r0 + _CH, :] * awx

            bi = bc = bgcx = bgcy = blgw = blgh = blab = None
            for j in range(N):
                ab = gt_ref[b, j, 4]
                gcx = gt_ref[b, j, 5]
                gcy = gt_ref[b, j, 6]
                lgw = gt_ref[b, j, 7]
                lgh = gt_ref[b, j, 8]
                lab = gt_ref[b, j, 9]
                wy = wy_all[r0:r0 + _CH, j:j + 1]
                inter = wy * wxs[j]
                c = area + ab
                if j == 0:
                    bi, bc = inter, c
                    bgcx = jnp.full_like(inter, gcx)
                    bgcy = jnp.full_like(inter, gcy)
                    blgw = jnp.full_like(inter, lgw)
                    blgh = jnp.full_like(inter, lgh)
                    blab = jnp.full_like(inter, lab)
                else:
                    cond = inter * bc > bi * c
                    bgcx = jnp.where(cond, gcx, bgcx)
                    bgcy = jnp.where(cond, gcy, bgcy)
                    blgw = jnp.where(cond, lgw, blgw)
                    blgh = jnp.where(cond, lgh, blgh)
                    blab = jnp.where(cond, lab, blab)
                    bi = jnp.where(cond, inter, bi)
                    bc = jnp.where(cond, c, bc)

            pos = (3.0 * bi) >= bc
            neg = (13.0 * bi) < (3.0 * bc)
            posf = pos.astype(jnp.float32)

            o = pred_ref[0, 8 * a + 4, rows, :]
            objl = (jnp.maximum(o, 0.0) - o * posf
                    + jnp.log1p(jnp.exp(-jnp.abs(o))))
            v_obj += objl * posf
            v_np += posf
            v_neg += neg.astype(jnp.float32)
            nv_ref[a, rows, :] = jnp.where(neg, objl, -1.0)

            x0 = pred_ref[0, 8 * a + 5, rows, :]
            x1 = pred_ref[0, 8 * a + 6, rows, :]
            x2 = pred_ref[0, 8 * a + 7, rows, :]
            m = jnp.maximum(jnp.maximum(x0, x1), x2)
            lse = m + jnp.log(jnp.exp(x0 - m) + jnp.exp(x1 - m)
                              + jnp.exp(x2 - m))
            xsel = jnp.where(blab == 1.0, x1,
                             jnp.where(blab == 2.0, x2, x0))
            v_ce += (lse - xsel) * posf

            tx = (bgcx - acx) * ivaw
            ty = (bgcy - acy[r0:r0 + _CH, :]) * ivah[r0:r0 + _CH, :]
            tw = blgw - law
            th = blgh - lah[r0:r0 + _CH, :]

            def _sl1(ch_off, t):
                d = pred_ref[0, 8 * a + ch_off, rows, :] - t
                ad = jnp.abs(d)
                return jnp.where(ad < 1.0, 0.5 * d * d, ad - 0.5)

            ll = _sl1(0, tx) + _sl1(1, ty) + _sl1(2, tw) + _sl1(3, th)
            v_ll += ll * posf

    sum_obj_pos = jnp.sum(v_obj)
    num_pos = jnp.sum(v_np)
    neg_cnt = jnp.sum(v_neg)
    sum_ce = jnp.sum(v_ce)
    sum_ll = jnp.sum(v_ll)

    k = jnp.minimum(neg_cnt, 3.0 * jnp.maximum(num_pos, 1.0))

    def _bis(_, carry):
        lo, hi = carry
        mid = lo + lax.div(hi - lo, jnp.int32(2))
        cnt = jnp.float32(0.0)
        for a in range(A):
            vi = lax.bitcast_convert_type(nv_ref[a, :, :], jnp.int32)
            cnt += jnp.sum(jnp.where(vi >= mid, 1.0, 0.0))
        geq = cnt >= k
        return (jnp.where(geq, mid, lo), jnp.where(geq, hi, mid))

    lo, _ = lax.fori_loop(0, 31, _bis, (jnp.int32(0), jnp.int32(_INF_BITS)))

    tplane = lax.bitcast_convert_type(
        jnp.full((8, 128), lo, dtype=jnp.int32), jnp.float32)
    t = jnp.max(tplane)
    cnt_gt = jnp.float32(0.0)
    s_gt = jnp.float32(0.0)
    for a in range(A):
        v = nv_ref[a, :, :]
        vi = lax.bitcast_convert_type(v, jnp.int32)
        m_gt = vi > lo
        cnt_gt += jnp.sum(jnp.where(m_gt, 1.0, 0.0))
        s_gt += jnp.sum(jnp.where(m_gt, v, 0.0))
    topk_sum = s_gt + (k - cnt_gt) * t

    acc_ref[0] += sum_obj_pos + topk_sum
    acc_ref[1] += num_pos + k
    acc_ref[2] += sum_ce
    acc_ref[3] += sum_ll
    acc_ref[4] += num_pos

    @pl.when(b == pl.num_programs(0) - 1)
    def _fin():
        obj_n = jnp.maximum(acc_ref[1], 1.0)
        pos_n = jnp.maximum(acc_ref[4], 1.0)
        l_obj = acc_ref[0] / obj_n
        l_cls = acc_ref[2] / pos_n
        l_loc = acc_ref[3] / pos_n
        l_tot = l_obj + l_cls + 2.0 * l_loc
        li = lax.broadcasted_iota(jnp.int32, (8, 128), 1)
        out_ref[:, :] = jnp.where(
            li == 0, l_obj,
            jnp.where(li == 1, l_cls,
                      jnp.where(li == 2, l_loc,
                                jnp.where(li == 3, l_tot, 0.0))))


def kernel(predictions, target_boxes, target_labels, anchors):
    B, ch, H, W = predictions.shape
    A = ch // (5 + _NCLS)
    N = target_boxes.shape[1]
    NP = 32

    anc = anchors.reshape(H, W, A, 4)
    ax = jnp.stack([anc[0, :, :, 0].T, anc[0, :, :, 2].T])
    ay = jnp.stack([anc[:, 0, :, 1], anc[:, 0, :, 3]])
    ay = jnp.pad(ay, ((0, 0), (0, 0), (0, 16 - A)))

    x1 = target_boxes[..., 0]
    y1 = target_boxes[..., 1]
    x2 = target_boxes[..., 2]
    y2 = target_boxes[..., 3]
    area = jnp.maximum(x2 - x1, 0.0) * jnp.maximum(y2 - y1, 0.0)
    gw = jnp.maximum(x2 - x1, _EPS)
    gh = jnp.maximum(y2 - y1, _EPS)
    gt = jnp.stack([x1, y1, x2, y2, area,
                    x1 + 0.5 * gw, y1 + 0.5 * gh,
                    jnp.log(gw + _EPS), jnp.log(gh + _EPS),
                    target_labels.astype(jnp.float32)], axis=-1)
    by = jnp.stack([y1, y2], axis=1)
    by = jnp.pad(by, ((0, 0), (0, 0), (0, NP - N)))

    out = pl.pallas_call(
        _loss_kernel,
        grid=(B,),
        in_specs=[
            pl.BlockSpec(memory_space=pltpu.SMEM),
            pl.BlockSpec((2, A, W), lambda b: (0, 0, 0)),
            pl.BlockSpec((2, H, 16), lambda b: (0, 0, 0)),
            pl.BlockSpec((1, 2, NP), lambda b: (b, 0, 0)),
            pl.BlockSpec((1, ch, H, W), lambda b: (b, 0, 0, 0)),
        ],
        out_specs=pl.BlockSpec((8, 128), lambda b: (0, 0)),
        out_shape=jax.ShapeDtypeStruct((8, 128), jnp.float32),
        scratch_shapes=[
            pltpu.VMEM((A, H, W), jnp.float32),
            pltpu.SMEM((8,), jnp.float32),
        ],
    )(gt, ax, ay, by, predictions)
    return out[0, 0:4]


# speedup vs baseline: 1.1860x; 1.1860x over previous
import jax
import jax.numpy as jnp
from jax import lax
from jax.experimental import pallas as pl
from jax.experimental.pallas import tpu as pltpu

_NCLS = 3
_EPS = 1e-6
_INF_BITS = 0x7F800000
_CH = 16


def _loss_kernel(gt_ref, ax_ref, ay_ref, by_ref, pred_ref, out_ref,
                 nv_ref, acc_ref):
    b = pl.program_id(0)
    A = nv_ref.shape[0]
    H, W = pred_ref.shape[2], pred_ref.shape[3]
    N = gt_ref.shape[1]

    @pl.when(b == 0)
    def _init():
        for i in range(5):
            acc_ref[i] = 0.0

    v_obj = jnp.zeros((_CH, W), jnp.float32)
    v_np = jnp.zeros((_CH, W), jnp.float32)
    v_neg = jnp.zeros((_CH, W), jnp.float32)
    v_ce = jnp.zeros((_CH, W), jnp.float32)
    v_ll = jnp.zeros((_CH, W), jnp.float32)

    N = gt_ref.shape[1]
    g_bx1 = [gt_ref[b, j, 0] for j in range(N)]
    g_bx2 = [gt_ref[b, j, 2] for j in range(N)]
    g_ab = [gt_ref[b, j, 4] for j in range(N)]
    g_gcx = [gt_ref[b, j, 5] for j in range(N)]
    g_gcy = [gt_ref[b, j, 6] for j in range(N)]
    g_lgw = [gt_ref[b, j, 7] for j in range(N)]
    g_lgh = [gt_ref[b, j, 8] for j in range(N)]
    g_lab = [gt_ref[b, j, 9] for j in range(N)]

    for a in range(A):
        ax1 = ax_ref[0, a:a + 1, :]
        ax2 = ax_ref[1, a:a + 1, :]
        awx = jnp.maximum(ax2 - ax1, 0.0)
        aw = jnp.maximum(ax2 - ax1, _EPS)
        acx = ax1 + 0.5 * aw
        ivaw = 1.0 / (aw + _EPS)
        law = jnp.log(aw + _EPS)

        ay1 = ay_ref[0, :, a:a + 1]
        ay2 = ay_ref[1, :, a:a + 1]
        awy = jnp.maximum(ay2 - ay1, 0.0)
        ah = jnp.maximum(ay2 - ay1, _EPS)
        acy = ay1 + 0.5 * ah
        ivah = 1.0 / (ah + _EPS)
        lah = jnp.log(ah + _EPS)

        by1 = by_ref[0, 0:1, :]
        by2 = by_ref[0, 1:2, :]
        wy_all = jnp.maximum(jnp.minimum(ay2, by2) - jnp.maximum(ay1, by1),
                             0.0)

        for hc in range(H // _CH):
            r0 = hc * _CH
            rows = pl.ds(r0, _CH)
            area = awy[r0:r0 + _CH, :] * awx

            bi = bc = bgcx = bgcy = blgw = blgh = blab = None
            for j in range(N):
                ab = g_ab[j]
                gcx = g_gcx[j]
                gcy = g_gcy[j]
                lgw = g_lgw[j]
                lgh = g_lgh[j]
                lab = g_lab[j]
                wx = jnp.maximum(
                    jnp.minimum(ax2, g_bx2[j]) - jnp.maximum(ax1, g_bx1[j]),
                    0.0)
                wy = wy_all[r0:r0 + _CH, j:j + 1]
                inter = wy * wx
                c = area + ab
                if j == 0:
                    bi, bc = inter, c
                    bgcx = jnp.full_like(inter, gcx)
                    bgcy = jnp.full_like(inter, gcy)
                    blgw = jnp.full_like(inter, lgw)
                    blgh = jnp.full_like(inter, lgh)
                    blab = jnp.full_like(inter, lab)
                else:
                    cond = inter * bc > bi * c
                    bgcx = jnp.where(cond, gcx, bgcx)
                    bgcy = jnp.where(cond, gcy, bgcy)
                    blgw = jnp.where(cond, lgw, blgw)
                    blgh = jnp.where(cond, lgh, blgh)
                    blab = jnp.where(cond, lab, blab)
                    bi = jnp.where(cond, inter, bi)
                    bc = jnp.where(cond, c, bc)

            pos = (3.0 * bi) >= bc
            neg = (13.0 * bi) < (3.0 * bc)
            posf = pos.astype(jnp.float32)

            o = pred_ref[0, 8 * a + 4, rows, :]
            objl = (jnp.maximum(o, 0.0) - o * posf
                    + jnp.log1p(jnp.exp(-jnp.abs(o))))
            v_obj += objl * posf
            v_np += posf
            v_neg += neg.astype(jnp.float32)
            nv_ref[a, rows, :] = jnp.where(neg, objl, -1.0)

            x0 = pred_ref[0, 8 * a + 5, rows, :]
            x1 = pred_ref[0, 8 * a + 6, rows, :]
            x2 = pred_ref[0, 8 * a + 7, rows, :]
            m = jnp.maximum(jnp.maximum(x0, x1), x2)
            lse = m + jnp.log(jnp.exp(x0 - m) + jnp.exp(x1 - m)
                              + jnp.exp(x2 - m))
            xsel = jnp.where(blab == 1.0, x1,
                             jnp.where(blab == 2.0, x2, x0))
            v_ce += (lse - xsel) * posf

            tx = (bgcx - acx) * ivaw
            ty = (bgcy - acy[r0:r0 + _CH, :]) * ivah[r0:r0 + _CH, :]
            tw = blgw - law
            th = blgh - lah[r0:r0 + _CH, :]

            def _sl1(ch_off, t):
                d = pred_ref[0, 8 * a + ch_off, rows, :] - t
                ad = jnp.abs(d)
                return jnp.where(ad < 1.0, 0.5 * d * d, ad - 0.5)

            ll = _sl1(0, tx) + _sl1(1, ty) + _sl1(2, tw) + _sl1(3, th)
            v_ll += ll * posf

    sum_obj_pos = jnp.sum(v_obj)
    num_pos = jnp.sum(v_np)
    neg_cnt = jnp.sum(v_neg)
    sum_ce = jnp.sum(v_ce)
    sum_ll = jnp.sum(v_ll)

    k = jnp.minimum(neg_cnt, 3.0 * jnp.maximum(num_pos, 1.0))

    def _bis(_, carry):
        lo, hi = carry
        mid = lo + lax.div(hi - lo, jnp.int32(2))
        cnt = jnp.float32(0.0)
        for a in range(A):
            vi = lax.bitcast_convert_type(nv_ref[a, :, :], jnp.int32)
            cnt += jnp.sum(jnp.where(vi >= mid, 1.0, 0.0))
        geq = cnt >= k
        return (jnp.where(geq, mid, lo), jnp.where(geq, hi, mid))

    lo, _ = lax.fori_loop(0, 31, _bis, (jnp.int32(0), jnp.int32(_INF_BITS)))

    tplane = lax.bitcast_convert_type(
        jnp.full((8, 128), lo, dtype=jnp.int32), jnp.float32)
    t = jnp.max(tplane)
    cnt_gt = jnp.float32(0.0)
    s_gt = jnp.float32(0.0)
    for a in range(A):
        v = nv_ref[a, :, :]
        vi = lax.bitcast_convert_type(v, jnp.int32)
        m_gt = vi > lo
        cnt_gt += jnp.sum(jnp.where(m_gt, 1.0, 0.0))
        s_gt += jnp.sum(jnp.where(m_gt, v, 0.0))
    topk_sum = s_gt + (k - cnt_gt) * t

    acc_ref[0] += sum_obj_pos + topk_sum
    acc_ref[1] += num_pos + k
    acc_ref[2] += sum_ce
    acc_ref[3] += sum_ll
    acc_ref[4] += num_pos

    @pl.when(b == pl.num_programs(0) - 1)
    def _fin():
        obj_n = jnp.maximum(acc_ref[1], 1.0)
        pos_n = jnp.maximum(acc_ref[4], 1.0)
        l_obj = acc_ref[0] / obj_n
        l_cls = acc_ref[2] / pos_n
        l_loc = acc_ref[3] / pos_n
        l_tot = l_obj + l_cls + 2.0 * l_loc
        li = lax.broadcasted_iota(jnp.int32, (8, 128), 1)
        out_ref[:, :] = jnp.where(
            li == 0, l_obj,
            jnp.where(li == 1, l_cls,
                      jnp.where(li == 2, l_loc,
                                jnp.where(li == 3, l_tot, 0.0))))


def kernel(predictions, target_boxes, target_labels, anchors):
    B, ch, H, W = predictions.shape
    A = ch // (5 + _NCLS)
    N = target_boxes.shape[1]
    NP = 32

    anc = anchors.reshape(H, W, A, 4)
    ax = jnp.stack([anc[0, :, :, 0].T, anc[0, :, :, 2].T])
    ay = jnp.stack([anc[:, 0, :, 1], anc[:, 0, :, 3]])
    ay = jnp.pad(ay, ((0, 0), (0, 0), (0, 16 - A)))

    x1 = target_boxes[..., 0]
    y1 = target_boxes[..., 1]
    x2 = target_boxes[..., 2]
    y2 = target_boxes[..., 3]
    area = jnp.maximum(x2 - x1, 0.0) * jnp.maximum(y2 - y1, 0.0)
    gw = jnp.maximum(x2 - x1, _EPS)
    gh = jnp.maximum(y2 - y1, _EPS)
    gt = jnp.stack([x1, y1, x2, y2, area,
                    x1 + 0.5 * gw, y1 + 0.5 * gh,
                    jnp.log(gw + _EPS), jnp.log(gh + _EPS),
                    target_labels.astype(jnp.float32)], axis=-1)
    by = jnp.stack([y1, y2], axis=1)
    by = jnp.pad(by, ((0, 0), (0, 0), (0, NP - N)))

    out = pl.pallas_call(
        _loss_kernel,
        grid=(B,),
        in_specs=[
            pl.BlockSpec(memory_space=pltpu.SMEM),
            pl.BlockSpec((2, A, W), lambda b: (0, 0, 0)),
            pl.BlockSpec((2, H, 16), lambda b: (0, 0, 0)),
            pl.BlockSpec((1, 2, NP), lambda b: (b, 0, 0)),
            pl.BlockSpec((1, ch, H, W), lambda b: (b, 0, 0, 0)),
        ],
        out_specs=pl.BlockSpec((8, 128), lambda b: (0, 0)),
        out_shape=jax.ShapeDtypeStruct((8, 128), jnp.float32),
        scratch_shapes=[
            pltpu.VMEM((A, H, W), jnp.float32),
            pltpu.SMEM((8,), jnp.float32),
        ],
    )(gt, ax, ay, by, predictions)
    return out[0, 0:4]


# speedup vs baseline: 1.2391x; 1.0448x over previous
import jax
import jax.numpy as jnp
from jax import lax
from jax.experimental import pallas as pl
from jax.experimental.pallas import tpu as pltpu

_NCLS = 3
_EPS = 1e-6
_INF_BITS = 0x7F800000
_CH = 32


def _loss_kernel(gt_ref, ax_ref, ay_ref, by_ref, pred_ref, out_ref,
                 nv_ref, acc_ref):
    b = pl.program_id(0)
    A = nv_ref.shape[0]
    H, W = pred_ref.shape[2], pred_ref.shape[3]
    N = gt_ref.shape[1]

    @pl.when(b == 0)
    def _init():
        for i in range(5):
            acc_ref[i] = 0.0

    v_obj = jnp.zeros((_CH, W), jnp.float32)
    v_np = jnp.zeros((_CH, W), jnp.float32)
    v_neg = jnp.zeros((_CH, W), jnp.float32)
    v_ce = jnp.zeros((_CH, W), jnp.float32)
    v_ll = jnp.zeros((_CH, W), jnp.float32)

    N = gt_ref.shape[1]
    g_bx1 = [gt_ref[b, j, 0] for j in range(N)]
    g_bx2 = [gt_ref[b, j, 2] for j in range(N)]
    g_ab = [gt_ref[b, j, 4] for j in range(N)]
    g_gcx = [gt_ref[b, j, 5] for j in range(N)]
    g_gcy = [gt_ref[b, j, 6] for j in range(N)]
    g_lgw = [gt_ref[b, j, 7] for j in range(N)]
    g_lgh = [gt_ref[b, j, 8] for j in range(N)]
    g_lab = [gt_ref[b, j, 9] for j in range(N)]

    for a in range(A):
        ax1 = ax_ref[0, a:a + 1, :]
        ax2 = ax_ref[1, a:a + 1, :]
        awx = jnp.maximum(ax2 - ax1, 0.0)
        aw = jnp.maximum(ax2 - ax1, _EPS)
        acx = ax1 + 0.5 * aw
        ivaw = 1.0 / (aw + _EPS)
        law = jnp.log(aw + _EPS)

        ay1 = ay_ref[0, :, a:a + 1]
        ay2 = ay_ref[1, :, a:a + 1]
        awy = jnp.maximum(ay2 - ay1, 0.0)
        ah = jnp.maximum(ay2 - ay1, _EPS)
        acy = ay1 + 0.5 * ah
        ivah = 1.0 / (ah + _EPS)
        lah = jnp.log(ah + _EPS)

        by1 = by_ref[0, 0:1, :]
        by2 = by_ref[0, 1:2, :]
        wy_all = jnp.maximum(jnp.minimum(ay2, by2) - jnp.maximum(ay1, by1),
                             0.0)

        for hc in range(H // _CH):
            r0 = hc * _CH
            rows = pl.ds(r0, _CH)
            area = awy[r0:r0 + _CH, :] * awx

            bi = bc = bgcx = bgcy = blgw = blgh = blab = None
            for j in range(N):
                ab = g_ab[j]
                gcx = g_gcx[j]
                gcy = g_gcy[j]
                lgw = g_lgw[j]
                lgh = g_lgh[j]
                lab = g_lab[j]
                wx = jnp.maximum(
                    jnp.minimum(ax2, g_bx2[j]) - jnp.maximum(ax1, g_bx1[j]),
                    0.0)
                wy = wy_all[r0:r0 + _CH, j:j + 1]
                inter = wy * wx
                c = area + ab
                if j == 0:
                    bi, bc = inter, c
                    bgcx = jnp.full_like(inter, gcx)
                    bgcy = jnp.full_like(inter, gcy)
                    blgw = jnp.full_like(inter, lgw)
                    blgh = jnp.full_like(inter, lgh)
                    blab = jnp.full_like(inter, lab)
                else:
                    cond = inter * bc > bi * c
                    bgcx = jnp.where(cond, gcx, bgcx)
                    bgcy = jnp.where(cond, gcy, bgcy)
                    blgw = jnp.where(cond, lgw, blgw)
                    blgh = jnp.where(cond, lgh, blgh)
                    blab = jnp.where(cond, lab, blab)
                    bi = jnp.where(cond, inter, bi)
                    bc = jnp.where(cond, c, bc)

            pos = (3.0 * bi) >= bc
            neg = (13.0 * bi) < (3.0 * bc)
            posf = pos.astype(jnp.float32)

            o = pred_ref[0, 8 * a + 4, rows, :]
            objl = (jnp.maximum(o, 0.0) - o * posf
                    + jnp.log1p(jnp.exp(-jnp.abs(o))))
            v_obj += objl * posf
            v_np += posf
            v_neg += neg.astype(jnp.float32)
            nv_ref[a, rows, :] = jnp.where(neg, objl, -1.0)

            x0 = pred_ref[0, 8 * a + 5, rows, :]
            x1 = pred_ref[0, 8 * a + 6, rows, :]
            x2 = pred_ref[0, 8 * a + 7, rows, :]
            m = jnp.maximum(jnp.maximum(x0, x1), x2)
            lse = m + jnp.log(jnp.exp(x0 - m) + jnp.exp(x1 - m)
                              + jnp.exp(x2 - m))
            xsel = jnp.where(blab == 1.0, x1,
                             jnp.where(blab == 2.0, x2, x0))
            v_ce += (lse - xsel) * posf

            tx = (bgcx - acx) * ivaw
            ty = (bgcy - acy[r0:r0 + _CH, :]) * ivah[r0:r0 + _CH, :]
            tw = blgw - law
            th = blgh - lah[r0:r0 + _CH, :]

            def _sl1(ch_off, t):
                d = pred_ref[0, 8 * a + ch_off, rows, :] - t
                ad = jnp.abs(d)
                return jnp.where(ad < 1.0, 0.5 * d * d, ad - 0.5)

            ll = _sl1(0, tx) + _sl1(1, ty) + _sl1(2, tw) + _sl1(3, th)
            v_ll += ll * posf

    sum_obj_pos = jnp.sum(v_obj)
    num_pos = jnp.sum(v_np)
    neg_cnt = jnp.sum(v_neg)
    sum_ce = jnp.sum(v_ce)
    sum_ll = jnp.sum(v_ll)

    k = jnp.minimum(neg_cnt, 3.0 * jnp.maximum(num_pos, 1.0))

    def _bis(_, carry):
        lo, hi = carry
        mid = lo + lax.div(hi - lo, jnp.int32(2))
        cnt = jnp.float32(0.0)
        for a in range(A):
            vi = lax.bitcast_convert_type(nv_ref[a, :, :], jnp.int32)
            cnt += jnp.sum(jnp.where(vi >= mid, 1.0, 0.0))
        geq = cnt >= k
        return (jnp.where(geq, mid, lo), jnp.where(geq, hi, mid))

    lo, _ = lax.fori_loop(0, 31, _bis, (jnp.int32(0), jnp.int32(_INF_BITS)))

    tplane = lax.bitcast_convert_type(
        jnp.full((8, 128), lo, dtype=jnp.int32), jnp.float32)
    t = jnp.max(tplane)
    cnt_gt = jnp.float32(0.0)
    s_gt = jnp.float32(0.0)
    for a in range(A):
        v = nv_ref[a, :, :]
        vi = lax.bitcast_convert_type(v, jnp.int32)
        m_gt = vi > lo
        cnt_gt += jnp.sum(jnp.where(m_gt, 1.0, 0.0))
        s_gt += jnp.sum(jnp.where(m_gt, v, 0.0))
    topk_sum = s_gt + (k - cnt_gt) * t

    acc_ref[0] += sum_obj_pos + topk_sum
    acc_ref[1] += num_pos + k
    acc_ref[2] += sum_ce
    acc_ref[3] += sum_ll
    acc_ref[4] += num_pos

    @pl.when(b == pl.num_programs(0) - 1)
    def _fin():
        obj_n = jnp.maximum(acc_ref[1], 1.0)
        pos_n = jnp.maximum(acc_ref[4], 1.0)
        l_obj = acc_ref[0] / obj_n
        l_cls = acc_ref[2] / pos_n
        l_loc = acc_ref[3] / pos_n
        l_tot = l_obj + l_cls + 2.0 * l_loc
        li = lax.broadcasted_iota(jnp.int32, (8, 128), 1)
        out_ref[:, :] = jnp.where(
            li == 0, l_obj,
            jnp.where(li == 1, l_cls,
                      jnp.where(li == 2, l_loc,
                                jnp.where(li == 3, l_tot, 0.0))))


def kernel(predictions, target_boxes, target_labels, anchors):
    B, ch, H, W = predictions.shape
    A = ch // (5 + _NCLS)
    N = target_boxes.shape[1]
    NP = 32

    anc = anchors.reshape(H, W, A, 4)
    ax = jnp.stack([anc[0, :, :, 0].T, anc[0, :, :, 2].T])
    ay = jnp.stack([anc[:, 0, :, 1], anc[:, 0, :, 3]])
    ay = jnp.pad(ay, ((0, 0), (0, 0), (0, 16 - A)))

    x1 = target_boxes[..., 0]
    y1 = target_boxes[..., 1]
    x2 = target_boxes[..., 2]
    y2 = target_boxes[..., 3]
    area = jnp.maximum(x2 - x1, 0.0) * jnp.maximum(y2 - y1, 0.0)
    gw = jnp.maximum(x2 - x1, _EPS)
    gh = jnp.maximum(y2 - y1, _EPS)
    gt = jnp.stack([x1, y1, x2, y2, area,
                    x1 + 0.5 * gw, y1 + 0.5 * gh,
                    jnp.log(gw + _EPS), jnp.log(gh + _EPS),
                    target_labels.astype(jnp.float32)], axis=-1)
    by = jnp.stack([y1, y2], axis=1)
    by = jnp.pad(by, ((0, 0), (0, 0), (0, NP - N)))

    out = pl.pallas_call(
        _loss_kernel,
        grid=(B,),
        in_specs=[
            pl.BlockSpec(memory_space=pltpu.SMEM),
            pl.BlockSpec((2, A, W), lambda b: (0, 0, 0)),
            pl.BlockSpec((2, H, 16), lambda b: (0, 0, 0)),
            pl.BlockSpec((1, 2, NP), lambda b: (b, 0, 0)),
            pl.BlockSpec((1, ch, H, W), lambda b: (b, 0, 0, 0)),
        ],
        out_specs=pl.BlockSpec((8, 128), lambda b: (0, 0)),
        out_shape=jax.ShapeDtypeStruct((8, 128), jnp.float32),
        scratch_shapes=[
            pltpu.VMEM((A, H, W), jnp.float32),
            pltpu.SMEM((8,), jnp.float32),
        ],
    )(gt, ax, ay, by, predictions)
    return out[0, 0:4]


# speedup vs baseline: 1.2460x; 1.0056x over previous
import jax
import jax.numpy as jnp
from jax import lax
from jax.experimental import pallas as pl
from jax.experimental.pallas import tpu as pltpu

_NCLS = 3
_EPS = 1e-6
_INF_BITS = 0x7F800000
_CH = 64


def _loss_kernel(gt_ref, ax_ref, ay_ref, by_ref, pred_ref, out_ref,
                 nv_ref, acc_ref):
    b = pl.program_id(0)
    A = nv_ref.shape[0]
    H, W = pred_ref.shape[2], pred_ref.shape[3]
    N = gt_ref.shape[1]

    @pl.when(b == 0)
    def _init():
        for i in range(5):
            acc_ref[i] = 0.0

    v_obj = jnp.zeros((_CH, W), jnp.float32)
    v_np = jnp.zeros((_CH, W), jnp.float32)
    v_neg = jnp.zeros((_CH, W), jnp.float32)
    v_ce = jnp.zeros((_CH, W), jnp.float32)
    v_ll = jnp.zeros((_CH, W), jnp.float32)

    N = gt_ref.shape[1]
    g_bx1 = [gt_ref[b, j, 0] for j in range(N)]
    g_bx2 = [gt_ref[b, j, 2] for j in range(N)]
    g_ab = [gt_ref[b, j, 4] for j in range(N)]
    g_gcx = [gt_ref[b, j, 5] for j in range(N)]
    g_gcy = [gt_ref[b, j, 6] for j in range(N)]
    g_lgw = [gt_ref[b, j, 7] for j in range(N)]
    g_lgh = [gt_ref[b, j, 8] for j in range(N)]
    g_lab = [gt_ref[b, j, 9] for j in range(N)]

    for a in range(A):
        ax1 = ax_ref[0, a:a + 1, :]
        ax2 = ax_ref[1, a:a + 1, :]
        awx = jnp.maximum(ax2 - ax1, 0.0)
        aw = jnp.maximum(ax2 - ax1, _EPS)
        acx = ax1 + 0.5 * aw
        ivaw = 1.0 / (aw + _EPS)
        law = jnp.log(aw + _EPS)

        ay1 = ay_ref[0, :, a:a + 1]
        ay2 = ay_ref[1, :, a:a + 1]
        awy = jnp.maximum(ay2 - ay1, 0.0)
        ah = jnp.maximum(ay2 - ay1, _EPS)
        acy = ay1 + 0.5 * ah
        ivah = 1.0 / (ah + _EPS)
        lah = jnp.log(ah + _EPS)

        by1 = by_ref[0, 0:1, :]
        by2 = by_ref[0, 1:2, :]
        wy_all = jnp.maximum(jnp.minimum(ay2, by2) - jnp.maximum(ay1, by1),
                             0.0)

        for hc in range(H // _CH):
            r0 = hc * _CH
            rows = pl.ds(r0, _CH)
            area = awy[r0:r0 + _CH, :] * awx

            bi = bc = bgcx = bgcy = blgw = blgh = blab = None
            for j in range(N):
                ab = g_ab[j]
                gcx = g_gcx[j]
                gcy = g_gcy[j]
                lgw = g_lgw[j]
                lgh = g_lgh[j]
                lab = g_lab[j]
                wx = jnp.maximum(
                    jnp.minimum(ax2, g_bx2[j]) - jnp.maximum(ax1, g_bx1[j]),
                    0.0)
                wy = wy_all[r0:r0 + _CH, j:j + 1]
                inter = wy * wx
                c = area + ab
                if j == 0:
                    bi, bc = inter, c
                    bgcx = jnp.full_like(inter, gcx)
                    bgcy = jnp.full_like(inter, gcy)
                    blgw = jnp.full_like(inter, lgw)
                    blgh = jnp.full_like(inter, lgh)
                    blab = jnp.full_like(inter, lab)
                else:
                    cond = inter * bc > bi * c
                    bgcx = jnp.where(cond, gcx, bgcx)
                    bgcy = jnp.where(cond, gcy, bgcy)
                    blgw = jnp.where(cond, lgw, blgw)
                    blgh = jnp.where(cond, lgh, blgh)
                    blab = jnp.where(cond, lab, blab)
                    bi = jnp.where(cond, inter, bi)
                    bc = jnp.where(cond, c, bc)

            pos = (3.0 * bi) >= bc
            neg = (13.0 * bi) < (3.0 * bc)
            posf = pos.astype(jnp.float32)

            o = pred_ref[0, 8 * a + 4, rows, :]
            objl = (jnp.maximum(o, 0.0) - o * posf
                    + jnp.log1p(jnp.exp(-jnp.abs(o))))
            v_obj += objl * posf
            v_np += posf
            v_neg += neg.astype(jnp.float32)
            nv_ref[a, rows, :] = jnp.where(neg, objl, -1.0)

            x0 = pred_ref[0, 8 * a + 5, rows, :]
            x1 = pred_ref[0, 8 * a + 6, rows, :]
            x2 = pred_ref[0, 8 * a + 7, rows, :]
            m = jnp.maximum(jnp.maximum(x0, x1), x2)
            lse = m + jnp.log(jnp.exp(x0 - m) + jnp.exp(x1 - m)
                              + jnp.exp(x2 - m))
            xsel = jnp.where(blab == 1.0, x1,
                             jnp.where(blab == 2.0, x2, x0))
            v_ce += (lse - xsel) * posf

            tx = (bgcx - acx) * ivaw
            ty = (bgcy - acy[r0:r0 + _CH, :]) * ivah[r0:r0 + _CH, :]
            tw = blgw - law
            th = blgh - lah[r0:r0 + _CH, :]

            def _sl1(ch_off, t):
                d = pred_ref[0, 8 * a + ch_off, rows, :] - t
                ad = jnp.abs(d)
                return jnp.where(ad < 1.0, 0.5 * d * d, ad - 0.5)

            ll = _sl1(0, tx) + _sl1(1, ty) + _sl1(2, tw) + _sl1(3, th)
            v_ll += ll * posf

    sum_obj_pos = jnp.sum(v_obj)
    num_pos = jnp.sum(v_np)
    neg_cnt = jnp.sum(v_neg)
    sum_ce = jnp.sum(v_ce)
    sum_ll = jnp.sum(v_ll)

    k = jnp.minimum(neg_cnt, 3.0 * jnp.maximum(num_pos, 1.0))

    def _bis(_, carry):
        lo, hi = carry
        mid = lo + lax.div(hi - lo, jnp.int32(2))
        cnt = jnp.float32(0.0)
        for a in range(A):
            vi = lax.bitcast_convert_type(nv_ref[a, :, :], jnp.int32)
            cnt += jnp.sum(jnp.where(vi >= mid, 1.0, 0.0))
        geq = cnt >= k
        return (jnp.where(geq, mid, lo), jnp.where(geq, hi, mid))

    lo, _ = lax.fori_loop(0, 31, _bis, (jnp.int32(0), jnp.int32(_INF_BITS)))

    tplane = lax.bitcast_convert_type(
        jnp.full((8, 128), lo, dtype=jnp.int32), jnp.float32)
    t = jnp.max(tplane)
    cnt_gt = jnp.float32(0.0)
    s_gt = jnp.float32(0.0)
    for a in range(A):
        v = nv_ref[a, :, :]
        vi = lax.bitcast_convert_type(v, jnp.int32)
        m_gt = vi > lo
        cnt_gt += jnp.sum(jnp.where(m_gt, 1.0, 0.0))
        s_gt += jnp.sum(jnp.where(m_gt, v, 0.0))
    topk_sum = s_gt + (k - cnt_gt) * t

    acc_ref[0] += sum_obj_pos + topk_sum
    acc_ref[1] += num_pos + k
    acc_ref[2] += sum_ce
    acc_ref[3] += sum_ll
    acc_ref[4] += num_pos

    @pl.when(b == pl.num_programs(0) - 1)
    def _fin():
        obj_n = jnp.maximum(acc_ref[1], 1.0)
        pos_n = jnp.maximum(acc_ref[4], 1.0)
        l_obj = acc_ref[0] / obj_n
        l_cls = acc_ref[2] / pos_n
        l_loc = acc_ref[3] / pos_n
        l_tot = l_obj + l_cls + 2.0 * l_loc
        li = lax.broadcasted_iota(jnp.int32, (8, 128), 1)
        out_ref[:, :] = jnp.where(
            li == 0, l_obj,
            jnp.where(li == 1, l_cls,
                      jnp.where(li == 2, l_loc,
                                jnp.where(li == 3, l_tot, 0.0))))


def kernel(predictions, target_boxes, target_labels, anchors):
    B, ch, H, W = predictions.shape
    A = ch // (5 + _NCLS)
    N = target_boxes.shape[1]
    NP = 32

    anc = anchors.reshape(H, W, A, 4)
    ax = jnp.stack([anc[0, :, :, 0].T, anc[0, :, :, 2].T])
    ay = jnp.stack([anc[:, 0, :, 1], anc[:, 0, :, 3]])
    ay = jnp.pad(ay, ((0, 0), (0, 0), (0, 16 - A)))

    x1 = target_boxes[..., 0]
    y1 = target_boxes[..., 1]
    x2 = target_boxes[..., 2]
    y2 = target_boxes[..., 3]
    area = jnp.maximum(x2 - x1, 0.0) * jnp.maximum(y2 - y1, 0.0)
    gw = jnp.maximum(x2 - x1, _EPS)
    gh = jnp.maximum(y2 - y1, _EPS)
    gt = jnp.stack([x1, y1, x2, y2, area,
                    x1 + 0.5 * gw, y1 + 0.5 * gh,
                    jnp.log(gw + _EPS), jnp.log(gh + _EPS),
                    target_labels.astype(jnp.float32)], axis=-1)
    by = jnp.stack([y1, y2], axis=1)
    by = jnp.pad(by, ((0, 0), (0, 0), (0, NP - N)))

    out = pl.pallas_call(
        _loss_kernel,
        grid=(B,),
        in_specs=[
            pl.BlockSpec(memory_space=pltpu.SMEM),
            pl.BlockSpec((2, A, W), lambda b: (0, 0, 0)),
            pl.BlockSpec((2, H, 16), lambda b: (0, 0, 0)),
            pl.BlockSpec((1, 2, NP), lambda b: (b, 0, 0)),
            pl.BlockSpec((1, ch, H, W), lambda b: (b, 0, 0, 0)),
        ],
        out_specs=pl.BlockSpec((8, 128), lambda b: (0, 0)),
        out_shape=jax.ShapeDtypeStruct((8, 128), jnp.float32),
        scratch_shapes=[
            pltpu.VMEM((A, H, W), jnp.float32),
            pltpu.SMEM((8,), jnp.float32),
        ],
    )(gt, ax, ay, by, predictions)
    return out[0, 0:4]


# speedup vs baseline: 1.3021x; 1.0450x over previous
import jax
import jax.numpy as jnp
from jax import lax
from jax.experimental import pallas as pl
from jax.experimental.pallas import tpu as pltpu

_NCLS = 3
_EPS = 1e-6
_INF_BITS = 0x7F800000


def _loss_kernel(gt_ref, ax_ref, ay_ref, pred_ref, out_ref, nv_ref, acc_ref):
    b = pl.program_id(0)
    A = nv_ref.shape[0]
    N = gt_ref.shape[1]

    @pl.when(b == 0)
    def _init():
        for i in range(5):
            acc_ref[i] = 0.0

    sum_obj_pos = jnp.float32(0.0)
    num_pos = jnp.float32(0.0)
    neg_cnt = jnp.float32(0.0)
    sum_ce = jnp.float32(0.0)
    sum_ll = jnp.float32(0.0)

    for a in range(A):
        ax1 = ax_ref[0, a:a + 1, :]
        ax2 = ax_ref[1, a:a + 1, :]
        ay1 = ay_ref[0, :, a:a + 1]
        ay2 = ay_ref[1, :, a:a + 1]
        awx = jnp.maximum(ax2 - ax1, 0.0)
        awy = jnp.maximum(ay2 - ay1, 0.0)
        area = awy * awx
        aw = jnp.maximum(ax2 - ax1, _EPS)
        ah = jnp.maximum(ay2 - ay1, _EPS)
        acx = ax1 + 0.5 * aw
        acy = ay1 + 0.5 * ah
        ivaw = 1.0 / (aw + _EPS)
        ivah = 1.0 / (ah + _EPS)
        law = jnp.log(aw + _EPS)
        lah = jnp.log(ah + _EPS)

        bi = bc = bgcx = bgcy = blgw = blgh = blab = None
        for j in range(N):
            bx1 = gt_ref[b, j, 0]
            by1 = gt_ref[b, j, 1]
            bx2 = gt_ref[b, j, 2]
            by2 = gt_ref[b, j, 3]
            ab = gt_ref[b, j, 4]
            gcx = gt_ref[b, j, 5]
            gcy = gt_ref[b, j, 6]
            lgw = gt_ref[b, j, 7]
            lgh = gt_ref[b, j, 8]
            lab = gt_ref[b, j, 9]
            wx = jnp.maximum(jnp.minimum(ax2, bx2) - jnp.maximum(ax1, bx1), 0.0)
            wy = jnp.maximum(jnp.minimum(ay2, by2) - jnp.maximum(ay1, by1), 0.0)
            inter = wy * wx
            c = area + ab
            if j == 0:
                bi, bc = inter, c
                bgcx = jnp.full_like(inter, gcx)
                bgcy = jnp.full_like(inter, gcy)
                blgw = jnp.full_like(inter, lgw)
                blgh = jnp.full_like(inter, lgh)
                blab = jnp.full_like(inter, lab)
            else:
                cond = inter * bc > bi * c
                bgcx = jnp.where(cond, gcx, bgcx)
                bgcy = jnp.where(cond, gcy, bgcy)
                blgw = jnp.where(cond, lgw, blgw)
                blgh = jnp.where(cond, lgh, blgh)
                blab = jnp.where(cond, lab, blab)
                bi = jnp.where(cond, inter, bi)
                bc = jnp.where(cond, c, bc)

        pos = (3.0 * bi) >= bc
        neg = (13.0 * bi) < (3.0 * bc)
        posf = pos.astype(jnp.float32)

        o = pred_ref[0, 8 * a + 4, :, :]
        objl = jnp.maximum(o, 0.0) - o * posf + jnp.log1p(jnp.exp(-jnp.abs(o)))
        sum_obj_pos += jnp.sum(objl * posf)
        num_pos += jnp.sum(posf)
        neg_cnt += jnp.sum(neg.astype(jnp.float32))
        nv_ref[a, :, :] = jnp.where(neg, objl, -1.0)

        x0 = pred_ref[0, 8 * a + 5, :, :]
        x1 = pred_ref[0, 8 * a + 6, :, :]
        x2 = pred_ref[0, 8 * a + 7, :, :]
        m = jnp.maximum(jnp.maximum(x0, x1), x2)
        lse = m + jnp.log(jnp.exp(x0 - m) + jnp.exp(x1 - m) + jnp.exp(x2 - m))
        xsel = jnp.where(blab == 1.0, x1, jnp.where(blab == 2.0, x2, x0))
        sum_ce += jnp.sum((lse - xsel) * posf)

        tx = (bgcx - acx) * ivaw
        ty = (bgcy - acy) * ivah
        tw = blgw - law
        th = blgh - lah

        def _sl1(ch_off, t):
            d = pred_ref[0, 8 * a + ch_off, :, :] - t
            ad = jnp.abs(d)
            return jnp.where(ad < 1.0, 0.5 * d * d, ad - 0.5)

        ll = _sl1(0, tx) + _sl1(1, ty) + _sl1(2, tw) + _sl1(3, th)
        sum_ll += jnp.sum(ll * posf)

    k = jnp.minimum(neg_cnt, 3.0 * jnp.maximum(num_pos, 1.0))

    def _bis4(_, carry):
        lo, hi = carry
        step = lax.div(hi - lo, jnp.int32(4))
        m1 = lo + step
        m2 = m1 + step
        m3 = m2 + step
        a1 = jnp.zeros((8, 128), jnp.float32)
        a2 = jnp.zeros((8, 128), jnp.float32)
        a3 = jnp.zeros((8, 128), jnp.float32)
        for a in range(A):
            for r in range(16):
                vi = lax.bitcast_convert_type(
                    nv_ref[a, 8 * r:8 * r + 8, :], jnp.int32)
                a1 += jnp.where(vi >= m1, 1.0, 0.0)
                a2 += jnp.where(vi >= m2, 1.0, 0.0)
                a3 += jnp.where(vi >= m3, 1.0, 0.0)
        c1 = jnp.sum(a1)
        c2 = jnp.sum(a2)
        c3 = jnp.sum(a3)
        lo2 = jnp.where(c3 >= k, m3,
                        jnp.where(c2 >= k, m2,
                                  jnp.where(c1 >= k, m1, lo)))
        hi2 = jnp.where(c1 < k, m1,
                        jnp.where(c2 < k, m2,
                                  jnp.where(c3 < k, m3, hi)))
        return lo2, hi2

    def _bis(_, carry):
        lo, hi = carry
        mid = lo + lax.div(hi - lo, jnp.int32(2))
        cnt = jnp.float32(0.0)
        for a in range(A):
            vi = lax.bitcast_convert_type(nv_ref[a, :, :], jnp.int32)
            cnt += jnp.sum(jnp.where(vi >= mid, 1.0, 0.0))
        geq = cnt >= k
        return (jnp.where(geq, mid, lo), jnp.where(geq, hi, mid))

    carry = lax.fori_loop(0, 16, _bis4, (jnp.int32(0), jnp.int32(_INF_BITS)))
    lo, _ = lax.fori_loop(0, 3, _bis, carry)

    tplane = lax.bitcast_convert_type(
        jnp.full((8, 128), lo, dtype=jnp.int32), jnp.float32)
    t = jnp.max(tplane)
    cnt_gt = jnp.float32(0.0)
    s_gt = jnp.float32(0.0)
    for a in range(A):
        v = nv_ref[a, :, :]
        vi = lax.bitcast_convert_type(v, jnp.int32)
        m_gt = vi > lo
        cnt_gt += jnp.sum(jnp.where(m_gt, 1.0, 0.0))
        s_gt += jnp.sum(jnp.where(m_gt, v, 0.0))
    topk_sum = s_gt + (k - cnt_gt) * t

    acc_ref[0] += sum_obj_pos + topk_sum
    acc_ref[1] += num_pos + k
    acc_ref[2] += sum_ce
    acc_ref[3] += sum_ll
    acc_ref[4] += num_pos

    @pl.when(b == pl.num_programs(0) - 1)
    def _fin():
        obj_n = jnp.maximum(acc_ref[1], 1.0)
        pos_n = jnp.maximum(acc_ref[4], 1.0)
        l_obj = acc_ref[0] / obj_n
        l_cls = acc_ref[2] / pos_n
        l_loc = acc_ref[3] / pos_n
        l_tot = l_obj + l_cls + 2.0 * l_loc
        li = lax.broadcasted_iota(jnp.int32, (8, 128), 1)
        out_ref[:, :] = jnp.where(
            li == 0, l_obj,
            jnp.where(li == 1, l_cls,
                      jnp.where(li == 2, l_loc,
                                jnp.where(li == 3, l_tot, 0.0))))


def kernel(predictions, target_boxes, target_labels, anchors):
    B, ch, H, W = predictions.shape
    A = ch // (5 + _NCLS)

    anc = anchors.reshape(H, W, A, 4)
    ax = jnp.stack([anc[0, :, :, 0].T, anc[0, :, :, 2].T])
    ay = jnp.stack([anc[:, 0, :, 1], anc[:, 0, :, 3]])
    ay = jnp.pad(ay, ((0, 0), (0, 0), (0, 16 - A)))

    x1 = target_boxes[..., 0]
    y1 = target_boxes[..., 1]
    x2 = target_boxes[..., 2]
    y2 = target_boxes[..., 3]
    area = jnp.maximum(x2 - x1, 0.0) * jnp.maximum(y2 - y1, 0.0)
    gw = jnp.maximum(x2 - x1, _EPS)
    gh = jnp.maximum(y2 - y1, _EPS)
    gt = jnp.stack([x1, y1, x2, y2, area,
                    x1 + 0.5 * gw, y1 + 0.5 * gh,
                    jnp.log(gw + _EPS), jnp.log(gh + _EPS),
                    target_labels.astype(jnp.float32)], axis=-1)

    out = pl.pallas_call(
        _loss_kernel,
        grid=(B,),
        in_specs=[
            pl.BlockSpec(memory_space=pltpu.SMEM),
            pl.BlockSpec((2, A, W), lambda b: (0, 0, 0)),
            pl.BlockSpec((2, H, 16), lambda b: (0, 0, 0)),
            pl.BlockSpec((1, ch, H, W), lambda b: (b, 0, 0, 0)),
        ],
        out_specs=pl.BlockSpec((8, 128), lambda b: (0, 0)),
        out_shape=jax.ShapeDtypeStruct((8, 128), jnp.float32),
        scratch_shapes=[
            pltpu.VMEM((A, H, W), jnp.float32),
            pltpu.SMEM((8,), jnp.float32),
        ],
    )(gt, ax, ay, predictions)
    return out[0, 0:4]


# speedup vs baseline: 1.3269x; 1.0190x over previous
import jax
import jax.numpy as jnp
from jax import lax
from jax.experimental import pallas as pl
from jax.experimental.pallas import tpu as pltpu

_NCLS = 3
_EPS = 1e-6
_INF_BITS = 0x7F800000


def _loss_kernel(gt_ref, ax_ref, ay_ref, pred_ref, out_ref, nv_ref, acc_ref):
    b = pl.program_id(0)
    A = nv_ref.shape[0]
    N = gt_ref.shape[1]

    @pl.when(b == 0)
    def _init():
        for i in range(5):
            acc_ref[i] = 0.0

    sum_obj_pos = jnp.float32(0.0)
    num_pos = jnp.float32(0.0)
    neg_cnt = jnp.float32(0.0)
    sum_ce = jnp.float32(0.0)
    sum_ll = jnp.float32(0.0)

    for a in range(A):
        ax1 = ax_ref[0, a:a + 1, :]
        ax2 = ax_ref[1, a:a + 1, :]
        ay1 = ay_ref[0, :, a:a + 1]
        ay2 = ay_ref[1, :, a:a + 1]
        awx = jnp.maximum(ax2 - ax1, 0.0)
        awy = jnp.maximum(ay2 - ay1, 0.0)
        area = awy * awx
        aw = jnp.maximum(ax2 - ax1, _EPS)
        ah = jnp.maximum(ay2 - ay1, _EPS)
        acx = ax1 + 0.5 * aw
        acy = ay1 + 0.5 * ah
        ivaw = 1.0 / (aw + _EPS)
        ivah = 1.0 / (ah + _EPS)
        law = jnp.log(aw + _EPS)
        lah = jnp.log(ah + _EPS)

        bi = bc = bgcx = bgcy = blgw = blgh = blab = None
        for j in range(N):
            bx1 = gt_ref[b, j, 0]
            by1 = gt_ref[b, j, 1]
            bx2 = gt_ref[b, j, 2]
            by2 = gt_ref[b, j, 3]
            ab = gt_ref[b, j, 4]
            gcx = gt_ref[b, j, 5]
            gcy = gt_ref[b, j, 6]
            lgw = gt_ref[b, j, 7]
            lgh = gt_ref[b, j, 8]
            lab = gt_ref[b, j, 9]
            wx = jnp.maximum(jnp.minimum(ax2, bx2) - jnp.maximum(ax1, bx1), 0.0)
            wy = jnp.maximum(jnp.minimum(ay2, by2) - jnp.maximum(ay1, by1), 0.0)
            inter = wy * wx
            c = area + ab
            if j == 0:
                bi, bc = inter, c
                bgcx = jnp.full_like(inter, gcx)
                bgcy = jnp.full_like(inter, gcy)
                blgw = jnp.full_like(inter, lgw)
                blgh = jnp.full_like(inter, lgh)
                blab = jnp.full_like(inter, lab)
            else:
                cond = inter * bc > bi * c
                bgcx = jnp.where(cond, gcx, bgcx)
                bgcy = jnp.where(cond, gcy, bgcy)
                blgw = jnp.where(cond, lgw, blgw)
                blgh = jnp.where(cond, lgh, blgh)
                blab = jnp.where(cond, lab, blab)
                bi = jnp.where(cond, inter, bi)
                bc = jnp.where(cond, c, bc)

        pos = (3.0 * bi) >= bc
        neg = (13.0 * bi) < (3.0 * bc)
        posf = pos.astype(jnp.float32)

        o = pred_ref[0, 8 * a + 4, :, :]
        objl = jnp.maximum(o, 0.0) - o * posf + jnp.log1p(jnp.exp(-jnp.abs(o)))
        sum_obj_pos += jnp.sum(objl * posf)
        num_pos += jnp.sum(posf)
        neg_cnt += jnp.sum(neg.astype(jnp.float32))
        nv_ref[a, :, :] = jnp.where(neg, objl, -1.0)

        x0 = pred_ref[0, 8 * a + 5, :, :]
        x1 = pred_ref[0, 8 * a + 6, :, :]
        x2 = pred_ref[0, 8 * a + 7, :, :]
        m = jnp.maximum(jnp.maximum(x0, x1), x2)
        lse = m + jnp.log(jnp.exp(x0 - m) + jnp.exp(x1 - m) + jnp.exp(x2 - m))
        xsel = jnp.where(blab == 1.0, x1, jnp.where(blab == 2.0, x2, x0))
        sum_ce += jnp.sum((lse - xsel) * posf)

        tx = (bgcx - acx) * ivaw
        ty = (bgcy - acy) * ivah
        tw = blgw - law
        th = blgh - lah

        def _sl1(ch_off, t):
            d = pred_ref[0, 8 * a + ch_off, :, :] - t
            ad = jnp.abs(d)
            return jnp.where(ad < 1.0, 0.5 * d * d, ad - 0.5)

        ll = _sl1(0, tx) + _sl1(1, ty) + _sl1(2, tw) + _sl1(3, th)
        sum_ll += jnp.sum(ll * posf)

    k = jnp.minimum(neg_cnt, 3.0 * jnp.maximum(num_pos, 1.0))

    def _bis(_, carry):
        lo, hi = carry
        mid = lo + lax.div(hi - lo, jnp.int32(2))
        cnt = jnp.float32(0.0)
        for a in range(A):
            vi = lax.bitcast_convert_type(nv_ref[a, :, :], jnp.int32)
            cnt += jnp.sum(jnp.where(vi >= mid, 1.0, 0.0))
        geq = cnt >= k
        return (jnp.where(geq, mid, lo), jnp.where(geq, hi, mid))

    lo, _ = lax.fori_loop(0, 31, _bis, (jnp.int32(0), jnp.int32(_INF_BITS)))

    tplane = lax.bitcast_convert_type(
        jnp.full((8, 128), lo, dtype=jnp.int32), jnp.float32)
    t = jnp.max(tplane)
    cnt_gt = jnp.float32(0.0)
    s_gt = jnp.float32(0.0)
    for a in range(A):
        v = nv_ref[a, :, :]
        vi = lax.bitcast_convert_type(v, jnp.int32)
        m_gt = vi > lo
        cnt_gt += jnp.sum(jnp.where(m_gt, 1.0, 0.0))
        s_gt += jnp.sum(jnp.where(m_gt, v, 0.0))
    topk_sum = s_gt + (k - cnt_gt) * t

    acc_ref[0] += sum_obj_pos + topk_sum
    acc_ref[1] += num_pos + k
    acc_ref[2] += sum_ce
    acc_ref[3] += sum_ll
    acc_ref[4] += num_pos

    @pl.when(b == pl.num_programs(0) - 1)
    def _fin():
        obj_n = jnp.maximum(acc_ref[1], 1.0)
        pos_n = jnp.maximum(acc_ref[4], 1.0)
        l_obj = acc_ref[0] / obj_n
        l_cls = acc_ref[2] / pos_n
        l_loc = acc_ref[3] / pos_n
        l_tot = l_obj + l_cls + 2.0 * l_loc
        li = lax.broadcasted_iota(jnp.int32, (8, 128), 1)
        out_ref[:, :] = jnp.where(
            li == 0, l_obj,
            jnp.where(li == 1, l_cls,
                      jnp.where(li == 2, l_loc,
                                jnp.where(li == 3, l_tot, 0.0))))


def kernel(predictions, target_boxes, target_labels, anchors):
    B, ch, H, W = predictions.shape
    A = ch // (5 + _NCLS)

    anc = anchors.reshape(H, W, A, 4)
    ax = jnp.stack([anc[0, :, :, 0].T, anc[0, :, :, 2].T])
    ay = jnp.stack([anc[:, 0, :, 1], anc[:, 0, :, 3]])
    ay = jnp.pad(ay, ((0, 0), (0, 0), (0, 16 - A)))

    x1 = target_boxes[..., 0]
    y1 = target_boxes[..., 1]
    x2 = target_boxes[..., 2]
    y2 = target_boxes[..., 3]
    area = jnp.maximum(x2 - x1, 0.0) * jnp.maximum(y2 - y1, 0.0)
    gw = jnp.maximum(x2 - x1, _EPS)
    gh = jnp.maximum(y2 - y1, _EPS)
    gt = jnp.stack([x1, y1, x2, y2, area,
                    x1 + 0.5 * gw, y1 + 0.5 * gh,
                    jnp.log(gw + _EPS), jnp.log(gh + _EPS),
                    target_labels.astype(jnp.float32)], axis=-1)

    out = pl.pallas_call(
        _loss_kernel,
        grid=(B,),
        in_specs=[
            pl.BlockSpec(memory_space=pltpu.SMEM),
            pl.BlockSpec((2, A, W), lambda b: (0, 0, 0)),
            pl.BlockSpec((2, H, 16), lambda b: (0, 0, 0)),
            pl.BlockSpec((1, ch, H, W), lambda b: (b, 0, 0, 0)),
        ],
        out_specs=pl.BlockSpec((8, 128), lambda b: (0, 0)),
        out_shape=jax.ShapeDtypeStruct((8, 128), jnp.float32),
        scratch_shapes=[
            pltpu.VMEM((A, H, W), jnp.float32),
            pltpu.SMEM((8,), jnp.float32),
        ],
    )(gt, ax, ay, predictions)
    return out[0, 0:4]
